# Initial kernel scaffold; baseline (speedup 1.0000x reference)
import functools

import jax
import jax.numpy as jnp
from jax import lax
from jax.experimental import pallas as pl
from jax.experimental.pallas import tpu as pltpu
from jax.experimental.pallas import tpu_sc as plsc

BATCH = 16384
HIST = 50
DIM = 64
LANES = 16
NUM_CORES = 2
NUM_SUBCORES = 16
NUM_WORKERS = NUM_CORES * NUM_SUBCORES
ROWS_PER_WORKER = BATCH // NUM_WORKERS
CHUNK = 16
IDX_PER_CHUNK = CHUNK * HIST
NUM_CHUNKS = ROWS_PER_WORKER // CHUNK


def _encoder_kernel(x_hbm, tab_hbm, out_hbm, idx_v, rows_v, acc_v, sem):
    wid = lax.axis_index("s") * NUM_CORES + lax.axis_index("c")
    base = wid * ROWS_PER_WORKER

    @pl.loop(0, NUM_CHUNKS)
    def _(ch):
        row0 = base + ch * CHUNK
        pltpu.sync_copy(x_hbm.at[pl.ds(row0 * HIST, IDX_PER_CHUNK)], idx_v)
        pltpu.async_copy(tab_hbm.at[idx_v], rows_v, sem).wait()

        @pl.loop(0, CHUNK)
        def _(c):
            for d in range(DIM // LANES):
                sl = pl.ds(d * LANES, LANES)
                acc = rows_v[c * HIST, sl]
                for l in range(1, HIST):
                    acc = acc + rows_v[c * HIST + l, sl]
                acc_v[c, sl] = acc

        pltpu.sync_copy(acc_v, out_hbm.at[pl.ds(row0, CHUNK)])


def kernel(x, table):
    xf = x.reshape(-1).astype(jnp.int32)
    mesh = plsc.VectorSubcoreMesh(core_axis_name="c", subcore_axis_name="s")
    run = functools.partial(
        pl.kernel,
        out_type=jax.ShapeDtypeStruct((BATCH, DIM), jnp.float32),
        mesh=mesh,
        scratch_types=[
            pltpu.VMEM((IDX_PER_CHUNK,), jnp.int32),
            pltpu.VMEM((IDX_PER_CHUNK, DIM), jnp.float32),
            pltpu.VMEM((CHUNK, DIM), jnp.float32),
            pltpu.SemaphoreType.DMA,
        ],
    )(_encoder_kernel)
    return run(xf, table)


# speedup vs baseline: 2.3490x; 2.3490x over previous
import functools

import jax
import jax.numpy as jnp
from jax import lax
from jax.experimental import pallas as pl
from jax.experimental.pallas import tpu as pltpu
from jax.experimental.pallas import tpu_sc as plsc

BATCH = 16384
HIST = 50
DIM = 64
LANES = 16
NUM_CORES = 2
NUM_SUBCORES = 16
NUM_WORKERS = NUM_CORES * NUM_SUBCORES
ROWS_PER_WORKER = BATCH // NUM_WORKERS
CHUNK = 16
IDX_PER_CHUNK = CHUNK * HIST
NUM_CHUNKS = ROWS_PER_WORKER // CHUNK


def _encoder_kernel(x_hbm, tab_hbm, out_hbm, idx_v, rows_v, acc_v, sem):
    wid = lax.axis_index("s") * NUM_CORES + lax.axis_index("c")
    base = wid * ROWS_PER_WORKER

    @pl.loop(0, NUM_CHUNKS)
    def _(ch):
        row0 = base + ch * CHUNK
        pltpu.sync_copy(x_hbm.at[pl.ds(row0 * HIST, IDX_PER_CHUNK)], idx_v)
        pltpu.async_copy(tab_hbm.at[idx_v], rows_v, sem).wait()

        @pl.loop(0, CHUNK)
        def _(c):
            for d in range(DIM // LANES):
                sl = pl.ds(d * LANES, LANES)
                acc = rows_v[c * HIST, sl]
                for l in range(1, HIST):
                    acc = acc + rows_v[c * HIST + l, sl]
                acc_v[c, sl] = acc

        pltpu.sync_copy(acc_v, out_hbm.at[pl.ds(row0, CHUNK)])


def kernel(x, table):
    xf = x.reshape(-1).astype(jnp.int32)
    mesh = plsc.VectorSubcoreMesh(core_axis_name="c", subcore_axis_name="s")
    run = functools.partial(
        pl.kernel,
        out_type=jax.ShapeDtypeStruct((BATCH, DIM), jnp.float32),
        mesh=mesh,
        scratch_types=[
            pltpu.VMEM((IDX_PER_CHUNK,), jnp.int32),
            pltpu.VMEM((IDX_PER_CHUNK, DIM), jnp.float32),
            pltpu.VMEM((CHUNK, DIM), jnp.float32),
            pltpu.SemaphoreType.DMA,
        ],
        compiler_params=pltpu.CompilerParams(use_tc_tiling_on_sc=False),
    )(_encoder_kernel)
    return run(xf, table)


# speedup vs baseline: 2.7480x; 1.1698x over previous
import functools

import jax
import jax.numpy as jnp
from jax import lax
from jax.experimental import pallas as pl
from jax.experimental.pallas import tpu as pltpu
from jax.experimental.pallas import tpu_sc as plsc

BATCH = 16384
HIST = 50
DIM = 64
LANES = 16
NUM_CORES = 2
NUM_SUBCORES = 16
NUM_WORKERS = NUM_CORES * NUM_SUBCORES
ROWS_PER_WORKER = BATCH // NUM_WORKERS
CHUNK = 16
IDX_PER_CHUNK = CHUNK * HIST
NUM_CHUNKS = ROWS_PER_WORKER // CHUNK


def _encoder_kernel(x_hbm, tab_hbm, out_hbm, idx0, idx1, rows0, rows1,
                    acc_v, sem0, sem1):
    wid = lax.axis_index("s") * NUM_CORES + lax.axis_index("c")
    base = wid * ROWS_PER_WORKER
    bufs = ((idx0, rows0, sem0), (idx1, rows1, sem1))

    def start_gather(ch, buf):
        idx_v, rows_v, sem = buf
        pltpu.sync_copy(
            x_hbm.at[pl.ds((base + ch * CHUNK) * HIST, IDX_PER_CHUNK)], idx_v)
        pltpu.async_copy(tab_hbm.at[idx_v], rows_v, sem)

    start_gather(0, bufs[0])

    @pl.loop(0, NUM_CHUNKS, step=2)
    def _(ch):
        for b in range(2):
            cur = ch + b
            idx_v, rows_v, sem = bufs[b]

            @pl.when(cur + 1 < NUM_CHUNKS)
            def _():
                start_gather(cur + 1, bufs[b ^ 1])

            pltpu.make_async_copy(tab_hbm.at[idx_v], rows_v, sem).wait()

            @pl.loop(0, CHUNK)
            def _(c):
                slices = [pl.ds(d * LANES, LANES) for d in range(DIM // LANES)]
                accs = [rows_v[c * HIST, sl] for sl in slices]
                for l in range(1, HIST):
                    for d, sl in enumerate(slices):
                        accs[d] = accs[d] + rows_v[c * HIST + l, sl]
                for d, sl in enumerate(slices):
                    acc_v[c, sl] = accs[d]

            pltpu.sync_copy(acc_v, out_hbm.at[pl.ds(base + cur * CHUNK, CHUNK)])


def kernel(x, table):
    xf = x.reshape(-1).astype(jnp.int32)
    mesh = plsc.VectorSubcoreMesh(core_axis_name="c", subcore_axis_name="s")
    run = functools.partial(
        pl.kernel,
        out_type=jax.ShapeDtypeStruct((BATCH, DIM), jnp.float32),
        mesh=mesh,
        scratch_types=[
            pltpu.VMEM((IDX_PER_CHUNK,), jnp.int32),
            pltpu.VMEM((IDX_PER_CHUNK,), jnp.int32),
            pltpu.VMEM((IDX_PER_CHUNK, DIM), jnp.float32),
            pltpu.VMEM((IDX_PER_CHUNK, DIM), jnp.float32),
            pltpu.VMEM((CHUNK, DIM), jnp.float32),
            pltpu.SemaphoreType.DMA,
            pltpu.SemaphoreType.DMA,
        ],
        compiler_params=pltpu.CompilerParams(use_tc_tiling_on_sc=False),
    )(_encoder_kernel)
    return run(xf, table)
